# Initial kernel scaffold; baseline (speedup 1.0000x reference)
import jax
import jax.numpy as jnp
from jax.experimental import pallas as pl


def kernel(atom_feats, atom_xyz, surf_xyz, W1, b1, g1, be1, W2, b2, g2, be2, W3, b3, g3, be3):
    raise NotImplementedError("write your pallas kernel here")


# speedup vs baseline: 9.0181x; 9.0181x over previous
import functools

import jax
import jax.numpy as jnp
from jax import lax
from jax.experimental import pallas as pl
from jax.experimental.pallas import tpu as pltpu
from jax.experimental.pallas import tpu_sc as plsc

BB, CC, NN, MM, KK = 4, 64, 2048, 8192, 16
MT = 256
MT2 = 512
CNT1 = float(BB * MM * KK)
CNT3 = float(BB * MM)


def _k1_body(surf_ref, atom_ref, gidx_ref, dist_ref, sums_ref):
    b = pl.program_id(0)
    t = pl.program_id(1)
    s = surf_ref[0]
    a = atom_ref[0]
    s2 = (s[:, 0:1] * s[:, 0:1] + s[:, 1:2] * s[:, 1:2]) + s[:, 2:3] * s[:, 2:3]
    a2 = (a[0:1, :] * a[0:1, :] + a[1:2, :] * a[1:2, :]) + a[2:3, :] * a[2:3, :]
    e = jnp.dot(_bf(s), _bf(a), preferred_element_type=jnp.float32)
    d2 = (s2 + a2) - 2.0 * e
    iota_f = lax.broadcasted_iota(jnp.int32, (MT, NN), 1).astype(jnp.float32)
    inf = jnp.float32(float("inf"))
    bigf = jnp.float32(NN)
    idx_cols, dist_cols = [], []
    work = d2
    for _ in range(KK):
        mval = jnp.min(work, axis=1, keepdims=True)
        mi = jnp.where(work == mval, iota_f, bigf)
        amin = jnp.min(mi, axis=1, keepdims=True)
        work = jnp.where(mi == amin, inf, work)
        idx_cols.append(amin)
        dist_cols.append(jnp.sqrt(jnp.maximum(mval, 1e-12)))
    idxb = jnp.concatenate(idx_cols, axis=1).astype(jnp.int32)
    distb = jnp.concatenate(dist_cols, axis=1)
    gidx_ref[0] = idxb + b * NN
    dist_ref[0] = distb

    lane = lax.broadcasted_iota(jnp.int32, (1, 128), 1)
    db = distb.astype(jnp.bfloat16).astype(jnp.float32)
    sd = jnp.sum(db)
    sdd = jnp.sum(db * db)
    row = jnp.where(lane == 0, sd, 0.0) + jnp.where(lane == 1, sdd, 0.0)

    @pl.when((b == 0) & (t == 0))
    def _():
        sums_ref[...] = jnp.zeros_like(sums_ref)
    sums_ref[...] += row


def _k1_call(surf_xyz, atom_t):
    return pl.pallas_call(
        _k1_body,
        grid=(BB, MM // MT),
        in_specs=[
            pl.BlockSpec((1, MT, 3), lambda b, t: (b, t, 0)),
            pl.BlockSpec((1, 3, NN), lambda b, t: (b, 0, 0)),
        ],
        out_specs=[
            pl.BlockSpec((1, MT, KK), lambda b, t: (b, t, 0)),
            pl.BlockSpec((1, MT, KK), lambda b, t: (b, t, 0)),
            pl.BlockSpec((1, 128), lambda b, t: (0, 0)),
        ],
        out_shape=[
            jax.ShapeDtypeStruct((BB, MM, KK), jnp.int32),
            jax.ShapeDtypeStruct((BB, MM, KK), jnp.float32),
            jax.ShapeDtypeStruct((1, 128), jnp.float32),
        ],
    )(surf_xyz, atom_t)


_NW = 32
_PERW = BB * MM * KK // _NW
_CH = 512


def _sc_gather_call(table, gidx_flat):
    mesh = plsc.VectorSubcoreMesh(core_axis_name="c", subcore_axis_name="s")

    @functools.partial(
        pl.kernel,
        mesh=mesh,
        out_type=jax.ShapeDtypeStruct((BB * MM * KK, 128), jnp.float32),
        scratch_types=[
            pltpu.VMEM((_CH,), jnp.int32),
            pltpu.VMEM((_CH, 128), jnp.float32),
            pltpu.SemaphoreType.DMA,
        ],
    )
    def k(table_hbm, gidx_hbm, neb_hbm, idx_v, rows_v, sem):
        wid = lax.axis_index("s") * 2 + lax.axis_index("c")

        def chunk_body(c, carry):
            base = wid * _PERW + c * _CH
            pltpu.sync_copy(gidx_hbm.at[pl.ds(base, _CH)], idx_v)
            pltpu.async_copy(table_hbm.at[idx_v], rows_v, sem).wait()
            pltpu.sync_copy(rows_v, neb_hbm.at[pl.ds(base, _CH)])
            return carry
        lax.fori_loop(0, _PERW // _CH, chunk_body, 0)

    return k(table, gidx_flat)


def _k3_body(neb_ref, dist_ref, g0_ref, gfd_ref, sf_ref):
    b = pl.program_id(0)
    t = pl.program_id(1)
    f2 = neb_ref[0][:, :, :CC].reshape(MT2 * KK, CC)
    dd = dist_ref[...]

    @pl.when((b == 0) & (t == 0))
    def _():
        g0_ref[...] = jnp.zeros_like(g0_ref)
        gfd_ref[...] = jnp.zeros_like(gfd_ref)
        sf_ref[...] = jnp.zeros_like(sf_ref)
    g0_ref[...] += lax.dot_general(_bf(f2), _bf(f2), (((0,), (0,)), ((), ())),
                                   preferred_element_type=jnp.float32)
    f2b = _bf(f2).astype(jnp.float32)
    ddb = _bf(dd).astype(jnp.float32)
    gfd_ref[...] += jnp.sum(ddb * f2b, axis=0, keepdims=True)
    sf_ref[...] += jnp.sum(f2b, axis=0, keepdims=True)


def _k3_call(neb4, dist_col):
    return pl.pallas_call(
        _k3_body,
        grid=(BB, MM // MT2),
        in_specs=[
            pl.BlockSpec((1, MT2, KK, 128), lambda b, t: (b, t, 0, 0)),
            pl.BlockSpec((MT2 * KK, 1), lambda b, t: (b * (MM // MT2) + t, 0)),
        ],
        out_specs=[
            pl.BlockSpec((CC, CC), lambda b, t: (0, 0)),
            pl.BlockSpec((1, CC), lambda b, t: (0, 0)),
            pl.BlockSpec((1, CC), lambda b, t: (0, 0)),
        ],
        out_shape=[
            jax.ShapeDtypeStruct((CC, CC), jnp.float32),
            jax.ShapeDtypeStruct((1, CC), jnp.float32),
            jax.ShapeDtypeStruct((1, CC), jnp.float32),
        ],
    )(neb4, dist_col)


def _k3b_body(g0_ref, gfd_ref, sf_ref, sums_ref, w1ft_ref, w1d_ref,
              b1_ref, g1_ref, be1_ref, scale_ref, shift_ref):
    lane = lax.broadcasted_iota(jnp.int32, (1, 128), 1)
    s = sums_ref[...]
    sd = jnp.sum(jnp.where(lane == 0, s, 0.0))
    sdd = jnp.sum(jnp.where(lane == 1, s, 0.0))
    w1ft = _bf(w1ft_ref[...]).astype(jnp.float32)
    w1d = _bf(w1d_ref[...]).astype(jnp.float32)
    b1 = b1_ref[...]

    mu_f = sf_ref[...] / CNT1
    mu_d = sd / CNT1
    ey = jnp.dot(mu_f, w1ft, preferred_element_type=jnp.float32) \
        + mu_d * w1d + b1
    u = jnp.dot(g0_ref[...], w1ft, preferred_element_type=jnp.float32)
    diag = jnp.sum(w1ft * u, axis=0, keepdims=True)
    cross = 2.0 * w1d * jnp.dot(gfd_ref[...], w1ft,
                                preferred_element_type=jnp.float32)
    dd = w1d * w1d * sdd
    ey2 = (diag + cross + dd) / CNT1 + 2.0 * b1 * (ey - b1) + b1 * b1
    var = ey2 - ey * ey
    scale = g1_ref[...] * lax.rsqrt(var + 1e-5)
    scale_ref[...] = scale
    shift_ref[...] = be1_ref[...] - scale * ey


def _k3b_call(g0, gfd, sf, sums, w1ft, w1d, b1r, g1r, be1r):
    return pl.pallas_call(
        _k3b_body,
        out_shape=[
            jax.ShapeDtypeStruct((1, CC), jnp.float32),
            jax.ShapeDtypeStruct((1, CC), jnp.float32),
        ],
    )(g0, gfd, sf, sums, w1ft, w1d, b1r, g1r, be1r)


def _lrelu(x):
    return jnp.where(x >= 0, x, 0.2 * x)


def _bf(x):
    return x.astype(jnp.bfloat16)


def _stage1(neb_ref, dist_ref, w1ft_ref, w1d_ref, b1_ref, sc1_ref, sh1_ref):
    f2 = neb_ref[0][:, :, :CC].reshape(MT2 * KK, CC)
    dd = dist_ref[...]
    y1 = jnp.dot(_bf(f2), _bf(w1ft_ref[...]),
                 preferred_element_type=jnp.float32)
    ddb = _bf(dd).astype(jnp.float32)
    w1db = _bf(w1d_ref[...]).astype(jnp.float32)
    y1 = y1 + ddb * w1db + b1_ref[...]
    x1 = _lrelu(y1 * sc1_ref[...] + sh1_ref[...])
    return x1


def _k4_body(neb_ref, dist_ref, sc1_ref, sh1_ref, w1ft_ref, w1d_ref, b1_ref,
             g1_ref, sum1_ref):
    b = pl.program_id(0)
    t = pl.program_id(1)
    x1_2 = _stage1(neb_ref, dist_ref, w1ft_ref, w1d_ref, b1_ref,
                   sc1_ref, sh1_ref)

    @pl.when((b == 0) & (t == 0))
    def _():
        g1_ref[...] = jnp.zeros_like(g1_ref)
        sum1_ref[...] = jnp.zeros_like(sum1_ref)
    g1_ref[...] += lax.dot_general(_bf(x1_2), _bf(x1_2),
                                   (((0,), (0,)), ((), ())),
                                   preferred_element_type=jnp.float32)
    sum1_ref[...] += jnp.sum(_bf(x1_2).astype(jnp.float32), axis=0,
                             keepdims=True)


def _k4_call(neb4, dist_col, scale1, shift1, w1ft, w1d, b1r):
    return pl.pallas_call(
        _k4_body,
        grid=(BB, MM // MT2),
        in_specs=[
            pl.BlockSpec((1, MT2, KK, 128), lambda b, t: (b, t, 0, 0)),
            pl.BlockSpec((MT2 * KK, 1), lambda b, t: (b * (MM // MT2) + t, 0)),
            pl.BlockSpec((1, CC), lambda b, t: (0, 0)),
            pl.BlockSpec((1, CC), lambda b, t: (0, 0)),
            pl.BlockSpec((CC, CC), lambda b, t: (0, 0)),
            pl.BlockSpec((1, CC), lambda b, t: (0, 0)),
            pl.BlockSpec((1, CC), lambda b, t: (0, 0)),
        ],
        out_specs=[
            pl.BlockSpec((CC, CC), lambda b, t: (0, 0)),
            pl.BlockSpec((1, CC), lambda b, t: (0, 0)),
        ],
        out_shape=[
            jax.ShapeDtypeStruct((CC, CC), jnp.float32),
            jax.ShapeDtypeStruct((1, CC), jnp.float32),
        ],
    )(neb4, dist_col, scale1, shift1, w1ft, w1d, b1r)


def _stats_body(g_ref, sum_ref, wt_ref, b_ref, gam_ref, bet_ref,
                scale_ref, shift_ref, *, cnt):
    wt = _bf(wt_ref[...]).astype(jnp.float32)
    b = b_ref[...]
    mu = sum_ref[...] / cnt
    ey = jnp.dot(mu, wt, preferred_element_type=jnp.float32) + b
    u = jnp.dot(g_ref[...], wt, preferred_element_type=jnp.float32)
    diag = jnp.sum(wt * u, axis=0, keepdims=True) / cnt
    ey2 = diag + 2.0 * b * (ey - b) + b * b
    var = ey2 - ey * ey
    scale = gam_ref[...] * lax.rsqrt(var + 1e-5)
    scale_ref[...] = scale
    shift_ref[...] = bet_ref[...] - scale * ey


def _stats_call(g, s, wt, br, gr, ber, cnt):
    return pl.pallas_call(
        functools.partial(_stats_body, cnt=cnt),
        out_shape=[
            jax.ShapeDtypeStruct((1, CC), jnp.float32),
            jax.ShapeDtypeStruct((1, CC), jnp.float32),
        ],
    )(g, s, wt, br, gr, ber)


def _k5_body(neb_ref, dist_ref, sc1_ref, sh1_ref, sc2_ref, sh2_ref,
             w1ft_ref, w1d_ref, b1_ref, w2t_ref, b2_ref,
             sc_ref, g2_ref, sum2_ref):
    b = pl.program_id(0)
    t = pl.program_id(1)
    x1_2 = _stage1(neb_ref, dist_ref, w1ft_ref, w1d_ref, b1_ref,
                   sc1_ref, sh1_ref)
    y2 = jnp.dot(_bf(x1_2), _bf(w2t_ref[...]),
                 preferred_element_type=jnp.float32) + b2_ref[...]
    x2 = _lrelu(y2 * sc2_ref[...] + sh2_ref[...])
    s1 = jnp.sum(x1_2.reshape(MT2, KK, CC), axis=1)
    s2 = jnp.sum(x2.reshape(MT2, KK, CC), axis=1)
    sc = jnp.concatenate([s1, s2], axis=1)
    sc_ref[0] = sc

    @pl.when((b == 0) & (t == 0))
    def _():
        g2_ref[...] = jnp.zeros_like(g2_ref)
        sum2_ref[...] = jnp.zeros_like(sum2_ref)
    g2_ref[...] += lax.dot_general(_bf(sc), _bf(sc), (((0,), (0,)), ((), ())),
                                   preferred_element_type=jnp.float32)
    sum2_ref[...] += jnp.sum(_bf(sc).astype(jnp.float32), axis=0,
                             keepdims=True)


def _k5_call(neb4, dist_col, scale1, shift1, scale2, shift2, w1ft, w1d, b1r,
             w2t, b2r):
    vec = pl.BlockSpec((1, CC), lambda b, t: (0, 0))
    return pl.pallas_call(
        _k5_body,
        grid=(BB, MM // MT2),
        in_specs=[
            pl.BlockSpec((1, MT2, KK, 128), lambda b, t: (b, t, 0, 0)),
            pl.BlockSpec((MT2 * KK, 1), lambda b, t: (b * (MM // MT2) + t, 0)),
            vec, vec, vec, vec,
            pl.BlockSpec((CC, CC), lambda b, t: (0, 0)),
            vec, vec,
            pl.BlockSpec((CC, CC), lambda b, t: (0, 0)),
            vec,
        ],
        out_specs=[
            pl.BlockSpec((1, MT2, 2 * CC), lambda b, t: (b, t, 0)),
            pl.BlockSpec((2 * CC, 2 * CC), lambda b, t: (0, 0)),
            pl.BlockSpec((1, 2 * CC), lambda b, t: (0, 0)),
        ],
        out_shape=[
            jax.ShapeDtypeStruct((BB, MM, 2 * CC), jnp.float32),
            jax.ShapeDtypeStruct((2 * CC, 2 * CC), jnp.float32),
            jax.ShapeDtypeStruct((1, 2 * CC), jnp.float32),
        ],
    )(neb4, dist_col, scale1, shift1, scale2, shift2, w1ft, w1d, b1r,
      w2t, b2r)


def _k5b_call(g2, sum2, w3t, b3r, g3r, be3r):
    return pl.pallas_call(
        functools.partial(_stats_body, cnt=CNT3),
        out_shape=[
            jax.ShapeDtypeStruct((1, CC), jnp.float32),
            jax.ShapeDtypeStruct((1, CC), jnp.float32),
        ],
    )(g2, sum2, w3t, b3r, g3r, be3r)


def _k6_body(sc_ref, w3t_ref, sc3_ref, sh3_ref, out_ref):
    sc = sc_ref[0]
    y3 = jnp.dot(_bf(sc), _bf(w3t_ref[...]),
                 preferred_element_type=jnp.float32)
    o = _lrelu(y3 * sc3_ref[...] + sh3_ref[...])
    out_ref[0] = jnp.transpose(o)


def _k6_call(sc, w3t, scale3, shift3):
    return pl.pallas_call(
        _k6_body,
        grid=(BB, MM // MT2),
        in_specs=[
            pl.BlockSpec((1, MT2, 2 * CC), lambda b, t: (b, t, 0)),
            pl.BlockSpec((2 * CC, CC), lambda b, t: (0, 0)),
            pl.BlockSpec((1, CC), lambda b, t: (0, 0)),
            pl.BlockSpec((1, CC), lambda b, t: (0, 0)),
        ],
        out_specs=pl.BlockSpec((1, CC, MT2), lambda b, t: (b, 0, t)),
        out_shape=jax.ShapeDtypeStruct((BB, CC, MM), jnp.float32),
    )(sc, w3t, scale3, shift3)


def kernel(atom_feats, atom_xyz, surf_xyz, W1, b1, g1, be1, W2, b2, g2, be2,
           W3, b3, g3, be3):
    atom_t = jnp.transpose(atom_xyz, (0, 2, 1))
    ftab = jnp.transpose(atom_feats, (0, 2, 1)).reshape(BB * NN, CC)
    w1ft = jnp.transpose(W1[:, :CC])
    w1d = W1[:, CC][None, :]
    w2t = jnp.transpose(W2)
    w3t = jnp.transpose(W3)
    b1r, g1r, be1r = b1[None, :], g1[None, :], be1[None, :]
    b2r, g2r, be2r = b2[None, :], g2[None, :], be2[None, :]
    b3r, g3r, be3r = b3[None, :], g3[None, :], be3[None, :]

    gidx, dist, sums = _k1_call(surf_xyz, atom_t)
    dist_col = dist.reshape(BB * MM * KK, 1)
    ftab_pad = jnp.pad(ftab, ((0, 0), (0, 128 - CC)))
    neb = _sc_gather_call(ftab_pad, gidx.reshape(-1))
    neb4 = neb.reshape(BB, MM, KK, 128)
    g0, gfd, sf = _k3_call(neb4, dist_col)
    scale1, shift1 = _k3b_call(g0, gfd, sf, sums, w1ft, w1d, b1r, g1r, be1r)
    g1m, sum1 = _k4_call(neb4, dist_col, scale1, shift1, w1ft, w1d, b1r)
    scale2, shift2 = _stats_call(g1m, sum1, w2t, b2r, g2r, be2r, CNT1)
    sc, g2m, sum2 = _k5_call(neb4, dist_col, scale1, shift1, scale2, shift2,
                             w1ft, w1d, b1r, w2t, b2r)
    scale3, shift3 = _k5b_call(g2m, sum2, w3t, b3r, g3r, be3r)
    return _k6_call(sc, w3t, scale3, shift3)


# speedup vs baseline: 9.0246x; 1.0007x over previous
import functools

import jax
import jax.numpy as jnp
from jax import lax
from jax.experimental import pallas as pl
from jax.experimental.pallas import tpu as pltpu
from jax.experimental.pallas import tpu_sc as plsc

BB, CC, NN, MM, KK = 4, 64, 2048, 8192, 16
MT = 256
MT2 = 512
CNT1 = float(BB * MM * KK)
CNT3 = float(BB * MM)


def _k1_body(surf_ref, atom_ref, gidx_ref, dist_ref, sums_ref):
    b = pl.program_id(0)
    t = pl.program_id(1)
    s = surf_ref[0]
    a = atom_ref[0]
    s2 = (s[:, 0:1] * s[:, 0:1] + s[:, 1:2] * s[:, 1:2]) + s[:, 2:3] * s[:, 2:3]
    a2 = (a[0:1, :] * a[0:1, :] + a[1:2, :] * a[1:2, :]) + a[2:3, :] * a[2:3, :]
    e = jnp.dot(_bf(s), _bf(a), preferred_element_type=jnp.float32)
    d2 = (s2 + a2) - 2.0 * e
    iota_f = lax.broadcasted_iota(jnp.int32, (MT, NN), 1).astype(jnp.float32)
    inf = jnp.float32(float("inf"))
    bigf = jnp.float32(NN)
    idx_cols, dist_cols = [], []
    work = d2
    for _ in range(KK):
        mval = jnp.min(work, axis=1, keepdims=True)
        mi = jnp.where(work == mval, iota_f, bigf)
        amin = jnp.min(mi, axis=1, keepdims=True)
        work = jnp.where(mi == amin, inf, work)
        idx_cols.append(amin)
        dist_cols.append(jnp.sqrt(jnp.maximum(mval, 1e-12)))
    idxb = jnp.concatenate(idx_cols, axis=1).astype(jnp.int32)
    distb = jnp.concatenate(dist_cols, axis=1)
    gidx_ref[0] = idxb + b * NN
    dist_ref[0] = distb

    lane = lax.broadcasted_iota(jnp.int32, (1, 128), 1)
    db = distb.astype(jnp.bfloat16).astype(jnp.float32)
    sd = jnp.sum(db)
    sdd = jnp.sum(db * db)
    row = jnp.where(lane == 0, sd, 0.0) + jnp.where(lane == 1, sdd, 0.0)

    @pl.when((b == 0) & (t == 0))
    def _():
        sums_ref[...] = jnp.zeros_like(sums_ref)
    sums_ref[...] += row


def _k1_call(surf_xyz, atom_t):
    return pl.pallas_call(
        _k1_body,
        grid=(BB, MM // MT),
        in_specs=[
            pl.BlockSpec((1, MT, 3), lambda b, t: (b, t, 0)),
            pl.BlockSpec((1, 3, NN), lambda b, t: (b, 0, 0)),
        ],
        out_specs=[
            pl.BlockSpec((1, MT, KK), lambda b, t: (b, t, 0)),
            pl.BlockSpec((1, MT, KK), lambda b, t: (b, t, 0)),
            pl.BlockSpec((1, 128), lambda b, t: (0, 0)),
        ],
        out_shape=[
            jax.ShapeDtypeStruct((BB, MM, KK), jnp.int32),
            jax.ShapeDtypeStruct((BB, MM, KK), jnp.float32),
            jax.ShapeDtypeStruct((1, 128), jnp.float32),
        ],
    )(surf_xyz, atom_t)


_NW = 32
_PERW = BB * MM * KK // _NW
_CH = 512


def _sc_gather_call(table, gidx_flat):
    mesh = plsc.VectorSubcoreMesh(core_axis_name="c", subcore_axis_name="s")

    @functools.partial(
        pl.kernel,
        mesh=mesh,
        out_type=jax.ShapeDtypeStruct((BB * MM * KK, 128), jnp.float32),
        scratch_types=[
            pltpu.VMEM((_CH,), jnp.int32),
            pltpu.VMEM((_CH, 128), jnp.float32),
            pltpu.SemaphoreType.DMA,
        ],
    )
    def k(table_hbm, gidx_hbm, neb_hbm, idx_v, rows_v, sem):
        wid = lax.axis_index("s") * 2 + lax.axis_index("c")

        def chunk_body(c, carry):
            base = wid * _PERW + c * _CH
            pltpu.sync_copy(gidx_hbm.at[pl.ds(base, _CH)], idx_v)
            pltpu.async_copy(table_hbm.at[idx_v], rows_v, sem).wait()
            pltpu.sync_copy(rows_v, neb_hbm.at[pl.ds(base, _CH)])
            return carry
        lax.fori_loop(0, _PERW // _CH, chunk_body, 0)

    return k(table, gidx_flat)


def _y1(neb_ref, dist_ref, w1ft_ref, w1d_ref, b1_ref):
    f2 = neb_ref[0][:, :, :CC].reshape(MT2 * KK, CC)
    dd = dist_ref[...]
    y1 = jnp.dot(_bf(f2), _bf(w1ft_ref[...]),
                 preferred_element_type=jnp.float32)
    ddb = _bf(dd).astype(jnp.float32)
    w1db = _bf(w1d_ref[...]).astype(jnp.float32)
    return y1 + ddb * w1db + b1_ref[...]


def _k3_body(neb_ref, dist_ref, w1ft_ref, w1d_ref, b1_ref, s1_ref, s2_ref):
    b = pl.program_id(0)
    t = pl.program_id(1)
    y1 = _y1(neb_ref, dist_ref, w1ft_ref, w1d_ref, b1_ref)

    @pl.when((b == 0) & (t == 0))
    def _():
        s1_ref[...] = jnp.zeros_like(s1_ref)
        s2_ref[...] = jnp.zeros_like(s2_ref)
    s1_ref[...] += jnp.sum(y1, axis=0, keepdims=True)
    s2_ref[...] += jnp.sum(y1 * y1, axis=0, keepdims=True)


def _k3_call(neb4, dist_col, w1ft, w1d, b1r):
    vec = pl.BlockSpec((1, CC), lambda b, t: (0, 0))
    return pl.pallas_call(
        _k3_body,
        grid=(BB, MM // MT2),
        in_specs=[
            pl.BlockSpec((1, MT2, KK, 128), lambda b, t: (b, t, 0, 0)),
            pl.BlockSpec((MT2 * KK, 1), lambda b, t: (b * (MM // MT2) + t, 0)),
            pl.BlockSpec((CC, CC), lambda b, t: (0, 0)),
            vec, vec,
        ],
        out_specs=[
            pl.BlockSpec((1, CC), lambda b, t: (0, 0)),
            pl.BlockSpec((1, CC), lambda b, t: (0, 0)),
        ],
        out_shape=[
            jax.ShapeDtypeStruct((1, CC), jnp.float32),
            jax.ShapeDtypeStruct((1, CC), jnp.float32),
        ],
    )(neb4, dist_col, w1ft, w1d, b1r)


def _mstats_body(s1_ref, s2_ref, gam_ref, bet_ref, scale_ref, shift_ref, *,
                 cnt):
    mean = s1_ref[...] / cnt
    var = s2_ref[...] / cnt - mean * mean
    scale = gam_ref[...] * lax.rsqrt(var + 1e-5)
    scale_ref[...] = scale
    shift_ref[...] = bet_ref[...] - scale * mean


def _mstats_call(s1, s2, gr, ber, cnt):
    return pl.pallas_call(
        functools.partial(_mstats_body, cnt=cnt),
        out_shape=[
            jax.ShapeDtypeStruct((1, CC), jnp.float32),
            jax.ShapeDtypeStruct((1, CC), jnp.float32),
        ],
    )(s1, s2, gr, ber)


def _lrelu(x):
    return jnp.where(x >= 0, x, 0.2 * x)


def _bf(x):
    return x.astype(jnp.bfloat16)


def _stage1(neb_ref, dist_ref, w1ft_ref, w1d_ref, b1_ref, sc1_ref, sh1_ref):
    f2 = neb_ref[0][:, :, :CC].reshape(MT2 * KK, CC)
    dd = dist_ref[...]
    y1 = jnp.dot(_bf(f2), _bf(w1ft_ref[...]),
                 preferred_element_type=jnp.float32)
    ddb = _bf(dd).astype(jnp.float32)
    w1db = _bf(w1d_ref[...]).astype(jnp.float32)
    y1 = y1 + ddb * w1db + b1_ref[...]
    x1 = _lrelu(y1 * sc1_ref[...] + sh1_ref[...])
    return x1


def _k4_body(neb_ref, dist_ref, sc1_ref, sh1_ref, w1ft_ref, w1d_ref, b1_ref,
             w2t_ref, b2_ref, s1_ref, s2_ref):
    b = pl.program_id(0)
    t = pl.program_id(1)
    x1_2 = _stage1(neb_ref, dist_ref, w1ft_ref, w1d_ref, b1_ref,
                   sc1_ref, sh1_ref)
    y2 = jnp.dot(_bf(x1_2), _bf(w2t_ref[...]),
                 preferred_element_type=jnp.float32) + b2_ref[...]

    @pl.when((b == 0) & (t == 0))
    def _():
        s1_ref[...] = jnp.zeros_like(s1_ref)
        s2_ref[...] = jnp.zeros_like(s2_ref)
    s1_ref[...] += jnp.sum(y2, axis=0, keepdims=True)
    s2_ref[...] += jnp.sum(y2 * y2, axis=0, keepdims=True)


def _k4_call(neb4, dist_col, scale1, shift1, w1ft, w1d, b1r, w2t, b2r):
    vec = pl.BlockSpec((1, CC), lambda b, t: (0, 0))
    return pl.pallas_call(
        _k4_body,
        grid=(BB, MM // MT2),
        in_specs=[
            pl.BlockSpec((1, MT2, KK, 128), lambda b, t: (b, t, 0, 0)),
            pl.BlockSpec((MT2 * KK, 1), lambda b, t: (b * (MM // MT2) + t, 0)),
            vec, vec,
            pl.BlockSpec((CC, CC), lambda b, t: (0, 0)),
            vec, vec,
            pl.BlockSpec((CC, CC), lambda b, t: (0, 0)),
            vec,
        ],
        out_specs=[
            pl.BlockSpec((1, CC), lambda b, t: (0, 0)),
            pl.BlockSpec((1, CC), lambda b, t: (0, 0)),
        ],
        out_shape=[
            jax.ShapeDtypeStruct((1, CC), jnp.float32),
            jax.ShapeDtypeStruct((1, CC), jnp.float32),
        ],
    )(neb4, dist_col, scale1, shift1, w1ft, w1d, b1r, w2t, b2r)


def _k5_body(neb_ref, dist_ref, sc1_ref, sh1_ref, sc2_ref, sh2_ref,
             w1ft_ref, w1d_ref, b1_ref, w2t_ref, b2_ref, w3t_ref, b3_ref,
             sc_ref, s1_ref, s2_ref):
    b = pl.program_id(0)
    t = pl.program_id(1)
    x1_2 = _stage1(neb_ref, dist_ref, w1ft_ref, w1d_ref, b1_ref,
                   sc1_ref, sh1_ref)
    y2 = jnp.dot(_bf(x1_2), _bf(w2t_ref[...]),
                 preferred_element_type=jnp.float32) + b2_ref[...]
    x2 = _lrelu(y2 * sc2_ref[...] + sh2_ref[...])
    s1 = jnp.sum(x1_2.reshape(MT2, KK, CC), axis=1)
    s2 = jnp.sum(x2.reshape(MT2, KK, CC), axis=1)
    sc = jnp.concatenate([s1, s2], axis=1)
    sc_ref[0] = sc
    y3 = jnp.dot(_bf(sc), _bf(w3t_ref[...]),
                 preferred_element_type=jnp.float32) + b3_ref[...]

    @pl.when((b == 0) & (t == 0))
    def _():
        s1_ref[...] = jnp.zeros_like(s1_ref)
        s2_ref[...] = jnp.zeros_like(s2_ref)
    s1_ref[...] += jnp.sum(y3, axis=0, keepdims=True)
    s2_ref[...] += jnp.sum(y3 * y3, axis=0, keepdims=True)


def _k5_call(neb4, dist_col, scale1, shift1, scale2, shift2, w1ft, w1d, b1r,
             w2t, b2r, w3t, b3r):
    vec = pl.BlockSpec((1, CC), lambda b, t: (0, 0))
    return pl.pallas_call(
        _k5_body,
        grid=(BB, MM // MT2),
        in_specs=[
            pl.BlockSpec((1, MT2, KK, 128), lambda b, t: (b, t, 0, 0)),
            pl.BlockSpec((MT2 * KK, 1), lambda b, t: (b * (MM // MT2) + t, 0)),
            vec, vec, vec, vec,
            pl.BlockSpec((CC, CC), lambda b, t: (0, 0)),
            vec, vec,
            pl.BlockSpec((CC, CC), lambda b, t: (0, 0)),
            vec,
            pl.BlockSpec((2 * CC, CC), lambda b, t: (0, 0)),
            vec,
        ],
        out_specs=[
            pl.BlockSpec((1, MT2, 2 * CC), lambda b, t: (b, t, 0)),
            pl.BlockSpec((1, CC), lambda b, t: (0, 0)),
            pl.BlockSpec((1, CC), lambda b, t: (0, 0)),
        ],
        out_shape=[
            jax.ShapeDtypeStruct((BB, MM, 2 * CC), jnp.float32),
            jax.ShapeDtypeStruct((1, CC), jnp.float32),
            jax.ShapeDtypeStruct((1, CC), jnp.float32),
        ],
    )(neb4, dist_col, scale1, shift1, scale2, shift2, w1ft, w1d, b1r,
      w2t, b2r, w3t, b3r)


def _k6_body(sc_ref, w3t_ref, b3_ref, sc3_ref, sh3_ref, out_ref):
    sc = sc_ref[0]
    y3 = jnp.dot(_bf(sc), _bf(w3t_ref[...]),
                 preferred_element_type=jnp.float32) + b3_ref[...]
    o = _lrelu(y3 * sc3_ref[...] + sh3_ref[...])
    out_ref[0] = jnp.transpose(o)


def _k6_call(sc, w3t, b3r, scale3, shift3):
    return pl.pallas_call(
        _k6_body,
        grid=(BB, MM // MT2),
        in_specs=[
            pl.BlockSpec((1, MT2, 2 * CC), lambda b, t: (b, t, 0)),
            pl.BlockSpec((2 * CC, CC), lambda b, t: (0, 0)),
            pl.BlockSpec((1, CC), lambda b, t: (0, 0)),
            pl.BlockSpec((1, CC), lambda b, t: (0, 0)),
            pl.BlockSpec((1, CC), lambda b, t: (0, 0)),
        ],
        out_specs=pl.BlockSpec((1, CC, MT2), lambda b, t: (b, 0, t)),
        out_shape=jax.ShapeDtypeStruct((BB, CC, MM), jnp.float32),
    )(sc, w3t, b3r, scale3, shift3)


def kernel(atom_feats, atom_xyz, surf_xyz, W1, b1, g1, be1, W2, b2, g2, be2,
           W3, b3, g3, be3):
    atom_t = jnp.transpose(atom_xyz, (0, 2, 1))
    ftab = jnp.transpose(atom_feats, (0, 2, 1)).reshape(BB * NN, CC)
    w1ft = jnp.transpose(W1[:, :CC])
    w1d = W1[:, CC][None, :]
    w2t = jnp.transpose(W2)
    w3t = jnp.transpose(W3)
    b1r, g1r, be1r = b1[None, :], g1[None, :], be1[None, :]
    b2r, g2r, be2r = b2[None, :], g2[None, :], be2[None, :]
    b3r, g3r, be3r = b3[None, :], g3[None, :], be3[None, :]

    gidx, dist, sums = _k1_call(surf_xyz, atom_t)
    dist_col = dist.reshape(BB * MM * KK, 1)
    ftab_pad = jnp.pad(ftab, ((0, 0), (0, 128 - CC)))
    neb = _sc_gather_call(ftab_pad, gidx.reshape(-1))
    neb4 = neb.reshape(BB, MM, KK, 128)
    m1, m2 = _k3_call(neb4, dist_col, w1ft, w1d, b1r)
    scale1, shift1 = _mstats_call(m1, m2, g1r, be1r, CNT1)
    n1, n2 = _k4_call(neb4, dist_col, scale1, shift1, w1ft, w1d, b1r,
                      w2t, b2r)
    scale2, shift2 = _mstats_call(n1, n2, g2r, be2r, CNT1)
    sc, p1, p2 = _k5_call(neb4, dist_col, scale1, shift1, scale2, shift2,
                          w1ft, w1d, b1r, w2t, b2r, w3t, b3r)
    scale3, shift3 = _mstats_call(p1, p2, g3r, be3r, CNT3)
    return _k6_call(sc, w3t, b3r, scale3, shift3)
